# Initial kernel scaffold; baseline (speedup 1.0000x reference)
import jax
import jax.numpy as jnp
from jax.experimental import pallas as pl


def kernel(x, edge_index, edge_attr, batch, W1, b1, W2, b2, We, be, Wo, bo, Wf, bf):
    raise NotImplementedError("write your pallas kernel here")


# speedup vs baseline: 7.3427x; 7.3427x over previous
import functools

import jax
import jax.numpy as jnp
from jax import lax
from jax.experimental import pallas as pl
from jax.experimental.pallas import tpu as pltpu
from jax.experimental.pallas import tpu_sc as plsc

_N = 10000
_E = 320000
_G = 64
_NC = 2
_NS = 16
_NW = _NC * _NS
_EW = _E // _NW
_CH = 80
_NCH = _EW // _CH
_RT = _N // _NS

_DW = 128
_RB = 2000
_EB = 8000

_HI = lax.Precision.HIGHEST
_F32 = jnp.float32


def _dot(a, b):
    return jnp.dot(a, b, precision=_HI, preferred_element_type=_F32)


def _dotT(a, b):
    return lax.dot_general(a, b, (((0,), (0,)), ((), ())),
                           precision=_HI, preferred_element_type=_F32)


def _sc_mesh():
    return plsc.VectorSubcoreMesh(core_axis_name="c", subcore_axis_name="s")


def _make_agg():
    @functools.partial(
        pl.kernel,
        out_type=jax.ShapeDtypeStruct((_NW, _RT, 128), _F32),
        mesh=_sc_mesh(),
        scratch_types=[
            pltpu.VMEM_SHARED((_N, 128), _F32),
            pltpu.VMEM((_NCH, _CH), jnp.int32),
            pltpu.VMEM((_NCH, _CH), jnp.int32),
            pltpu.VMEM((_CH, 128), _F32),
            pltpu.SemaphoreType.DMA,
        ],
    )
    def agg(z_hbm, src_hbm, dst_hbm, zeros_hbm, out_hbm,
            acc, idx_s, idx_d, rows, sem):
        c = lax.axis_index("c")
        s = lax.axis_index("s")
        wid = s * _NC + c
        pltpu.sync_copy(zeros_hbm, acc.at[pl.ds(s * _RT, _RT)])
        pltpu.sync_copy(src_hbm.at[wid], idx_s)
        pltpu.sync_copy(dst_hbm.at[wid], idx_d)
        plsc.subcore_barrier()

        def step(j, carry):
            pltpu.async_copy(z_hbm.at[idx_s.at[j]], rows, sem).wait()
            pltpu.sync_copy(rows, acc.at[idx_d.at[j]], add=True)
            return carry

        lax.fori_loop(0, _NCH, step, 0)
        plsc.subcore_barrier()
        pltpu.sync_copy(acc.at[pl.ds(s * _RT, _RT)],
                        out_hbm.at[c * _NS + s])

    return agg


def _make_deg():
    @functools.partial(
        pl.kernel,
        out_type=jax.ShapeDtypeStruct((_NW, _RT, _DW), _F32),
        mesh=_sc_mesh(),
        scratch_types=[
            pltpu.VMEM_SHARED((_N, _DW), _F32),
            pltpu.VMEM((_NCH, _CH), jnp.int32),
            pltpu.VMEM((_CH, _DW), _F32),
            pltpu.SemaphoreType.DMA,
        ],
    )
    def deg(dst_hbm, zeros_hbm, ones_hbm, out_hbm, acc, idx_d, ones, sem):
        c = lax.axis_index("c")
        s = lax.axis_index("s")
        wid = s * _NC + c
        pltpu.sync_copy(zeros_hbm, acc.at[pl.ds(s * _RT, _RT)])
        pltpu.sync_copy(dst_hbm.at[wid], idx_d)
        pltpu.sync_copy(ones_hbm, ones)
        plsc.subcore_barrier()

        def step(j, carry):
            pltpu.sync_copy(ones, acc.at[idx_d.at[j]], add=True)
            return carry

        lax.fori_loop(0, _NCH, step, 0)
        plsc.subcore_barrier()
        pltpu.sync_copy(acc.at[pl.ds(s * _RT, _RT)],
                        out_hbm.at[c * _NS + s])

    return deg


def _make_head():
    @functools.partial(
        pl.kernel,
        out_type=jax.ShapeDtypeStruct((_E // _CH, _CH, 128), _F32),
        mesh=_sc_mesh(),
        scratch_types=[
            pltpu.VMEM((_NCH, _CH), jnp.int32),
            pltpu.VMEM((_NCH, _CH), jnp.int32),
            pltpu.VMEM((_CH, 128), _F32),
            pltpu.SemaphoreType.DMA,
        ],
    )
    def head(p_hbm, q_hbm, src_hbm, dst_hbm, out_hbm, idx_s, idx_d, buf, sem):
        c = lax.axis_index("c")
        s = lax.axis_index("s")
        wid = s * _NC + c
        pltpu.sync_copy(src_hbm.at[wid], idx_s)
        pltpu.sync_copy(dst_hbm.at[wid], idx_d)

        def step(j, carry):
            pltpu.async_copy(p_hbm.at[idx_s.at[j]], buf, sem).wait()
            pltpu.async_copy(q_hbm.at[idx_d.at[j]], buf, sem, add=True).wait()
            pltpu.sync_copy(buf, out_hbm.at[wid * _NCH + j])
            return carry

        lax.fori_loop(0, _NCH, step, 0)

    return head


_agg_sc = _make_agg()
_deg_sc = _make_deg()
_head_sc = _make_head()


def _mt_call(b2d):
    def body(b_ref, mt_ref):
        g = lax.broadcasted_iota(jnp.int32, (_N, _G), 1)
        mt_ref[...] = (b_ref[...] == g).astype(_F32)
    return pl.pallas_call(
        body, out_shape=jax.ShapeDtypeStruct((_N, _G), _F32))(b2d)


def _degsum_call(parts):
    def body(d_ref, out_ref):
        d = d_ref[...]
        out_ref[...] = d[:_N, 0:1] + d[_N:, 0:1]
    return pl.pallas_call(
        body, out_shape=jax.ShapeDtypeStruct((_N, 1), _F32))(parts)


def _g2_call(mt, h, wb):
    o = wb.shape[1]
    def body(mt_ref, h_ref, wb_ref, g2_ref):
        mt_ = mt_ref[...]
        sums = _dotT(mt_, h_ref[...])
        cnt = _dotT(mt_, jnp.ones((_N, 1), _F32))
        gp = sums / jnp.maximum(cnt, 1.0)
        g2_ref[...] = _dot(gp, wb_ref[...])
    return pl.pallas_call(
        body, out_shape=jax.ShapeDtypeStruct((_G, o), _F32))(mt, h, wb)


def _z_call(h, mt, deg, wt, g2, split):
    d = h.shape[1]
    o = wt.shape[1]
    nb = _N // _RB

    def body(h_ref, mt_ref, deg_ref, wt_ref, g2_ref, *outs):
        y = _dot(h_ref[...], wt_ref[...]) + _dot(mt_ref[...], g2_ref[...])
        z = y * lax.rsqrt(deg_ref[...] + 1.0)
        if split:
            outs[0][...] = z[:, :128]
            outs[1][...] = z[:, 128:]
        else:
            outs[0][...] = z

    in_specs = [
        pl.BlockSpec((_RB, d), lambda i: (i, 0)),
        pl.BlockSpec((_RB, _G), lambda i: (i, 0)),
        pl.BlockSpec((_RB, 1), lambda i: (i, 0)),
        pl.BlockSpec((d, o), lambda i: (0, 0)),
        pl.BlockSpec((_G, o), lambda i: (0, 0)),
    ]
    if split:
        out_shape = [jax.ShapeDtypeStruct((_N, 128), _F32)] * 2
        out_specs = [pl.BlockSpec((_RB, 128), lambda i: (i, 0))] * 2
    else:
        out_shape = jax.ShapeDtypeStruct((_N, o), _F32)
        out_specs = pl.BlockSpec((_RB, o), lambda i: (i, 0))
    return pl.pallas_call(body, grid=(nb,), in_specs=in_specs,
                          out_specs=out_specs, out_shape=out_shape)(
        h, mt, deg, wt, g2)


def _h1_call(agg_lo, agg_hi, zlo, zhi, deg, bias):
    nb = _N // _RB
    off = _N // _RB

    def body(a0, a1, c0, c1, zl, zh, dg, b_ref, out):
        nis = lax.rsqrt(dg[...] + 1.0)
        lo = (a0[...] + a1[...] + zl[...]) * nis
        hi = (c0[...] + c1[...] + zh[...]) * nis
        h = jnp.concatenate([lo, hi], axis=1) + b_ref[...]
        out[...] = jnp.maximum(h, 0.0)

    blk = lambda i: (i, 0)
    blk2 = lambda i: (i + off, 0)
    in_specs = [
        pl.BlockSpec((_RB, 128), blk),
        pl.BlockSpec((_RB, 128), blk2),
        pl.BlockSpec((_RB, 128), blk),
        pl.BlockSpec((_RB, 128), blk2),
        pl.BlockSpec((_RB, 128), blk),
        pl.BlockSpec((_RB, 128), blk),
        pl.BlockSpec((_RB, 1), blk),
        pl.BlockSpec((1, 256), lambda i: (0, 0)),
    ]
    return pl.pallas_call(
        body, grid=(nb,), in_specs=in_specs,
        out_specs=pl.BlockSpec((_RB, 256), blk),
        out_shape=jax.ShapeDtypeStruct((_N, 256), _F32))(
        agg_lo, agg_lo, agg_hi, agg_hi, zlo, zhi, deg, bias)


def _h2_call(agg, z, deg, bias):
    nb = _N // _RB
    off = _N // _RB

    def body(a0, a1, z_ref, dg, b_ref, out):
        nis = lax.rsqrt(dg[...] + 1.0)
        h = (a0[...] + a1[...] + z_ref[...]) * nis + b_ref[...]
        out[...] = jnp.maximum(h, 0.0)

    blk = lambda i: (i, 0)
    in_specs = [
        pl.BlockSpec((_RB, 128), blk),
        pl.BlockSpec((_RB, 128), lambda i: (i + off, 0)),
        pl.BlockSpec((_RB, 128), blk),
        pl.BlockSpec((_RB, 1), blk),
        pl.BlockSpec((1, 128), lambda i: (0, 0)),
    ]
    return pl.pallas_call(
        body, grid=(nb,), in_specs=in_specs,
        out_specs=pl.BlockSpec((_RB, 128), blk),
        out_shape=jax.ShapeDtypeStruct((_N, 128), _F32))(agg, agg, z, deg, bias)


def _r_call(ea, we, be, woc, bo):
    nb = _E // _EB

    def body(ea_ref, we_ref, be_ref, woc_ref, bo_ref, out_ref):
        e = jnp.maximum(_dot(ea_ref[...], we_ref[...]) + be_ref[...], 0.0)
        out_ref[...] = _dot(e, woc_ref[...]) + bo_ref[...]

    in_specs = [
        pl.BlockSpec((_EB, 16), lambda i: (i, 0)),
        pl.BlockSpec((16, 64), lambda i: (0, 0)),
        pl.BlockSpec((1, 64), lambda i: (0, 0)),
        pl.BlockSpec((64, 128), lambda i: (0, 0)),
        pl.BlockSpec((1, 128), lambda i: (0, 0)),
    ]
    return pl.pallas_call(
        body, grid=(nb,), in_specs=in_specs,
        out_specs=pl.BlockSpec((_EB, 128), lambda i: (i, 0)),
        out_shape=jax.ShapeDtypeStruct((_E, 128), _F32))(ea, we, be, woc, bo)


def _pq_call(h2, woa, wob):
    def body(h_ref, wa_ref, wb_ref, p_ref, q_ref):
        h = h_ref[...]
        p_ref[...] = _dot(h, wa_ref[...])
        q_ref[...] = _dot(h, wb_ref[...])
    return pl.pallas_call(
        body,
        out_shape=[jax.ShapeDtypeStruct((_N, 128), _F32)] * 2)(h2, woa, wob)


def _out_call(s_arr, r_arr, wf, bf):
    nb = _E // _EB

    def body(s_ref, r_ref, wf_ref, bf_ref, out_ref):
        ef = jnp.maximum(s_ref[...] + r_ref[...], 0.0)
        out_ref[...] = _dot(ef, wf_ref[...]) + bf_ref[...]

    in_specs = [
        pl.BlockSpec((_EB, 128), lambda i: (i, 0)),
        pl.BlockSpec((_EB, 128), lambda i: (i, 0)),
        pl.BlockSpec((128, 1), lambda i: (0, 0)),
        pl.BlockSpec((1, 1), lambda i: (0, 0)),
    ]
    return pl.pallas_call(
        body, grid=(nb,), in_specs=in_specs,
        out_specs=pl.BlockSpec((_EB, 1), lambda i: (i, 0)),
        out_shape=jax.ShapeDtypeStruct((_E, 1), _F32))(s_arr, r_arr, wf, bf)


def kernel(x, edge_index, edge_attr, batch, W1, b1, W2, b2, We, be, Wo, bo,
           Wf, bf):
    x = x.astype(_F32)
    ei = edge_index.astype(jnp.int32)
    src3 = ei[0].reshape(_NW, _NCH, _CH)
    dst3 = ei[1].reshape(_NW, _NCH, _CH)
    b2d = batch.astype(jnp.int32).reshape(_N, 1)
    zeros128 = jnp.zeros((_RT, 128), _F32)
    zeros8 = jnp.zeros((_RT, _DW), _F32)
    ones8 = jnp.ones((_CH, _DW), _F32)

    mt = _mt_call(b2d)
    degparts = _deg_sc(dst3, zeros8, ones8).reshape(2 * _N, _DW)
    deg = _degsum_call(degparts)

    g2a = _g2_call(mt, x, W1[128:])
    zlo, zhi = _z_call(x, mt, deg, W1[:128], g2a, split=True)
    agg_lo = _agg_sc(zlo, src3, dst3, zeros128).reshape(2 * _N, 128)
    agg_hi = _agg_sc(zhi, src3, dst3, zeros128).reshape(2 * _N, 128)
    h1 = _h1_call(agg_lo, agg_hi, zlo, zhi, deg, b1.reshape(1, 256))

    g2b = _g2_call(mt, h1, W2[256:])
    z2 = _z_call(h1, mt, deg, W2[:256], g2b, split=False)
    agg2 = _agg_sc(z2, src3, dst3, zeros128).reshape(2 * _N, 128)
    h2 = _h2_call(agg2, z2, deg, b2.reshape(1, 128))

    r_arr = _r_call(edge_attr.astype(_F32), We, be.reshape(1, 64),
                    Wo[256:], bo.reshape(1, 128))
    p_arr, q_arr = _pq_call(h2, Wo[:128], Wo[128:256])
    s_arr = _head_sc(p_arr, q_arr, src3, dst3).reshape(_E, 128)
    return _out_call(s_arr, r_arr, Wf, bf.reshape(1, 1))


# speedup vs baseline: 8.3898x; 1.1426x over previous
import functools

import jax
import jax.numpy as jnp
from jax import lax
from jax.experimental import pallas as pl
from jax.experimental.pallas import tpu as pltpu
from jax.experimental.pallas import tpu_sc as plsc

_N = 10000
_E = 320000
_G = 64
_NC = 2
_NS = 16
_NW = _NC * _NS
_EW = _E // _NW
_CH = 80
_NCH = _EW // _CH
_RT = _N // _NS
_NB = 5
_BCH = _NCH // _NB

_DW = 128
_RB = 2000
_EB = 8000

_HI = lax.Precision.HIGHEST
_F32 = jnp.float32


def _dot(a, b):
    return jnp.dot(a, b, precision=_HI, preferred_element_type=_F32)


def _dotT(a, b):
    return lax.dot_general(a, b, (((0,), (0,)), ((), ())),
                           precision=_HI, preferred_element_type=_F32)


def _sc_mesh():
    return plsc.VectorSubcoreMesh(core_axis_name="c", subcore_axis_name="s")


def _make_agg():
    @functools.partial(
        pl.kernel,
        out_type=jax.ShapeDtypeStruct((_NW, _RT, 128), _F32),
        mesh=_sc_mesh(),
        scratch_types=[
            pltpu.VMEM_SHARED((_N, 128), _F32),
            pltpu.VMEM((_BCH, _CH), jnp.int32),
            pltpu.VMEM((_BCH, _CH), jnp.int32),
            pltpu.VMEM((_CH, 128), _F32),
            pltpu.VMEM((_CH, 128), _F32),
            pltpu.SemaphoreType.DMA,
            pltpu.SemaphoreType.DMA,
        ],
    )
    def agg(z_hbm, src_hbm, dst_hbm, zeros_hbm, out_hbm,
            acc, idx_s, idx_d, rows0, rows1, sem0, sem1):
        c = lax.axis_index("c")
        s = lax.axis_index("s")
        wid = s * _NC + c
        pltpu.sync_copy(zeros_hbm, acc.at[pl.ds(s * _RT, _RT)])
        plsc.subcore_barrier()

        for b in range(_NB):
            pltpu.sync_copy(src_hbm.at[wid * _NB + b], idx_s)
            pltpu.sync_copy(dst_hbm.at[wid * _NB + b], idx_d)
            pltpu.async_copy(z_hbm.at[idx_s.at[0]], rows0, sem0)
            pltpu.async_copy(z_hbm.at[idx_s.at[1]], rows1, sem1)

            def step(i, carry):
                j = 2 * i
                pltpu.make_async_copy(z_hbm.at[idx_s.at[j]], rows0,
                                      sem0).wait()
                pltpu.sync_copy(rows0, acc.at[idx_d.at[j]], add=True)

                @pl.when(j + 2 < _BCH)
                def _():
                    pltpu.async_copy(z_hbm.at[idx_s.at[j + 2]], rows0, sem0)

                pltpu.make_async_copy(z_hbm.at[idx_s.at[j + 1]], rows1,
                                      sem1).wait()
                pltpu.sync_copy(rows1, acc.at[idx_d.at[j + 1]], add=True)

                @pl.when(j + 3 < _BCH)
                def _():
                    pltpu.async_copy(z_hbm.at[idx_s.at[j + 3]], rows1, sem1)

                return carry

            lax.fori_loop(0, _BCH // 2, step, 0)
            jt = _BCH - 1
            pltpu.make_async_copy(z_hbm.at[idx_s.at[jt]], rows0, sem0).wait()
            pltpu.sync_copy(rows0, acc.at[idx_d.at[jt]], add=True)

        plsc.subcore_barrier()
        pltpu.sync_copy(acc.at[pl.ds(s * _RT, _RT)],
                        out_hbm.at[c * _NS + s])

    return agg


def _make_deg():
    @functools.partial(
        pl.kernel,
        out_type=jax.ShapeDtypeStruct((_NW, _RT, _DW), _F32),
        mesh=_sc_mesh(),
        scratch_types=[
            pltpu.VMEM_SHARED((_N, _DW), _F32),
            pltpu.VMEM((_NCH, _CH), jnp.int32),
            pltpu.VMEM((_CH, _DW), _F32),
            pltpu.SemaphoreType.DMA,
        ],
    )
    def deg(dst_hbm, zeros_hbm, ones_hbm, out_hbm, acc, idx_d, ones, sem):
        c = lax.axis_index("c")
        s = lax.axis_index("s")
        wid = s * _NC + c
        pltpu.sync_copy(zeros_hbm, acc.at[pl.ds(s * _RT, _RT)])
        pltpu.sync_copy(dst_hbm.at[wid], idx_d)
        pltpu.sync_copy(ones_hbm, ones)
        plsc.subcore_barrier()

        def step(j, carry):
            pltpu.sync_copy(ones, acc.at[idx_d.at[j]], add=True)
            return carry

        lax.fori_loop(0, _NCH, step, 0)
        plsc.subcore_barrier()
        pltpu.sync_copy(acc.at[pl.ds(s * _RT, _RT)],
                        out_hbm.at[c * _NS + s])

    return deg


def _make_head():
    @functools.partial(
        pl.kernel,
        out_type=jax.ShapeDtypeStruct((_E // _CH, _CH, 128), _F32),
        mesh=_sc_mesh(),
        scratch_types=[
            pltpu.VMEM((_NCH, _CH), jnp.int32),
            pltpu.VMEM((_NCH, _CH), jnp.int32),
            pltpu.VMEM((_CH, 128), _F32),
            pltpu.VMEM((_CH, 128), _F32),
            pltpu.SemaphoreType.DMA,
            pltpu.SemaphoreType.DMA,
            pltpu.SemaphoreType.DMA,
            pltpu.SemaphoreType.DMA,
            pltpu.SemaphoreType.DMA,
        ],
    )
    def head(p_hbm, q_hbm, src_hbm, dst_hbm, out_hbm, idx_s, idx_d,
             buf0, buf1, sp0, sp1, sq, sw0, sw1):
        c = lax.axis_index("c")
        s = lax.axis_index("s")
        wid = s * _NC + c
        pltpu.sync_copy(src_hbm.at[wid], idx_s)
        pltpu.sync_copy(dst_hbm.at[wid], idx_d)

        pltpu.async_copy(p_hbm.at[idx_s.at[0]], buf0, sp0)
        pltpu.async_copy(p_hbm.at[idx_s.at[1]], buf1, sp1)

        def step(i, carry):
            j = 2 * i
            pltpu.make_async_copy(p_hbm.at[idx_s.at[j]], buf0, sp0).wait()
            pltpu.async_copy(q_hbm.at[idx_d.at[j]], buf0, sq, add=True).wait()
            w0 = pltpu.async_copy(buf0, out_hbm.at[wid * _NCH + j], sw0)

            pltpu.make_async_copy(p_hbm.at[idx_s.at[j + 1]], buf1, sp1).wait()
            pltpu.async_copy(q_hbm.at[idx_d.at[j + 1]], buf1, sq,
                             add=True).wait()
            w1 = pltpu.async_copy(buf1, out_hbm.at[wid * _NCH + j + 1], sw1)

            @pl.when(j + 2 < _NCH)
            def _():
                w0.wait()
                pltpu.async_copy(p_hbm.at[idx_s.at[j + 2]], buf0, sp0)

            @pl.when(j + 3 < _NCH)
            def _():
                w1.wait()
                pltpu.async_copy(p_hbm.at[idx_s.at[j + 3]], buf1, sp1)

            return carry

        lax.fori_loop(0, _NCH // 2, step, 0)
        jt = _NCH - 1
        pltpu.make_async_copy(buf1, out_hbm.at[wid * _NCH + jt - 1],
                              sw1).wait()
        pltpu.make_async_copy(p_hbm.at[idx_s.at[jt]], buf0, sp0).wait()
        pltpu.async_copy(q_hbm.at[idx_d.at[jt]], buf0, sq, add=True).wait()
        pltpu.sync_copy(buf0, out_hbm.at[wid * _NCH + jt])

    return head


_agg_sc = _make_agg()
_deg_sc = _make_deg()
_head_sc = _make_head()


def _mt_call(b2d):
    def body(b_ref, mt_ref):
        g = lax.broadcasted_iota(jnp.int32, (_N, _G), 1)
        mt_ref[...] = (b_ref[...] == g).astype(_F32)
    return pl.pallas_call(
        body, out_shape=jax.ShapeDtypeStruct((_N, _G), _F32))(b2d)


def _degsum_call(parts):
    def body(d_ref, out_ref):
        d = d_ref[...]
        out_ref[...] = d[:_N, 0:1] + d[_N:, 0:1]
    return pl.pallas_call(
        body, out_shape=jax.ShapeDtypeStruct((_N, 1), _F32))(parts)


def _g2_call(mt, h, wb):
    o = wb.shape[1]
    def body(mt_ref, h_ref, wb_ref, g2_ref):
        mt_ = mt_ref[...]
        sums = _dotT(mt_, h_ref[...])
        cnt = _dotT(mt_, jnp.ones((_N, 1), _F32))
        gp = sums / jnp.maximum(cnt, 1.0)
        g2_ref[...] = _dot(gp, wb_ref[...])
    return pl.pallas_call(
        body, out_shape=jax.ShapeDtypeStruct((_G, o), _F32))(mt, h, wb)


def _z_call(h, mt, deg, wt, g2, split):
    d = h.shape[1]
    o = wt.shape[1]
    nb = _N // _RB

    def body(h_ref, mt_ref, deg_ref, wt_ref, g2_ref, *outs):
        y = _dot(h_ref[...], wt_ref[...]) + _dot(mt_ref[...], g2_ref[...])
        z = y * lax.rsqrt(deg_ref[...] + 1.0)
        if split:
            outs[0][...] = z[:, :128]
            outs[1][...] = z[:, 128:]
        else:
            outs[0][...] = z

    in_specs = [
        pl.BlockSpec((_RB, d), lambda i: (i, 0)),
        pl.BlockSpec((_RB, _G), lambda i: (i, 0)),
        pl.BlockSpec((_RB, 1), lambda i: (i, 0)),
        pl.BlockSpec((d, o), lambda i: (0, 0)),
        pl.BlockSpec((_G, o), lambda i: (0, 0)),
    ]
    if split:
        out_shape = [jax.ShapeDtypeStruct((_N, 128), _F32)] * 2
        out_specs = [pl.BlockSpec((_RB, 128), lambda i: (i, 0))] * 2
    else:
        out_shape = jax.ShapeDtypeStruct((_N, o), _F32)
        out_specs = pl.BlockSpec((_RB, o), lambda i: (i, 0))
    return pl.pallas_call(body, grid=(nb,), in_specs=in_specs,
                          out_specs=out_specs, out_shape=out_shape)(
        h, mt, deg, wt, g2)


def _h1_call(agg_lo, agg_hi, zlo, zhi, deg, bias):
    nb = _N // _RB
    off = _N // _RB

    def body(a0, a1, c0, c1, zl, zh, dg, b_ref, out):
        nis = lax.rsqrt(dg[...] + 1.0)
        lo = (a0[...] + a1[...] + zl[...]) * nis
        hi = (c0[...] + c1[...] + zh[...]) * nis
        h = jnp.concatenate([lo, hi], axis=1) + b_ref[...]
        out[...] = jnp.maximum(h, 0.0)

    blk = lambda i: (i, 0)
    blk2 = lambda i: (i + off, 0)
    in_specs = [
        pl.BlockSpec((_RB, 128), blk),
        pl.BlockSpec((_RB, 128), blk2),
        pl.BlockSpec((_RB, 128), blk),
        pl.BlockSpec((_RB, 128), blk2),
        pl.BlockSpec((_RB, 128), blk),
        pl.BlockSpec((_RB, 128), blk),
        pl.BlockSpec((_RB, 1), blk),
        pl.BlockSpec((1, 256), lambda i: (0, 0)),
    ]
    return pl.pallas_call(
        body, grid=(nb,), in_specs=in_specs,
        out_specs=pl.BlockSpec((_RB, 256), blk),
        out_shape=jax.ShapeDtypeStruct((_N, 256), _F32))(
        agg_lo, agg_lo, agg_hi, agg_hi, zlo, zhi, deg, bias)


def _h2_call(agg, z, deg, bias):
    nb = _N // _RB
    off = _N // _RB

    def body(a0, a1, z_ref, dg, b_ref, out):
        nis = lax.rsqrt(dg[...] + 1.0)
        h = (a0[...] + a1[...] + z_ref[...]) * nis + b_ref[...]
        out[...] = jnp.maximum(h, 0.0)

    blk = lambda i: (i, 0)
    in_specs = [
        pl.BlockSpec((_RB, 128), blk),
        pl.BlockSpec((_RB, 128), lambda i: (i + off, 0)),
        pl.BlockSpec((_RB, 128), blk),
        pl.BlockSpec((_RB, 1), blk),
        pl.BlockSpec((1, 128), lambda i: (0, 0)),
    ]
    return pl.pallas_call(
        body, grid=(nb,), in_specs=in_specs,
        out_specs=pl.BlockSpec((_RB, 128), blk),
        out_shape=jax.ShapeDtypeStruct((_N, 128), _F32))(agg, agg, z, deg, bias)


def _r_call(ea, we, be, woc, bo):
    nb = _E // _EB

    def body(ea_ref, we_ref, be_ref, woc_ref, bo_ref, out_ref):
        e = jnp.maximum(_dot(ea_ref[...], we_ref[...]) + be_ref[...], 0.0)
        out_ref[...] = _dot(e, woc_ref[...]) + bo_ref[...]

    in_specs = [
        pl.BlockSpec((_EB, 16), lambda i: (i, 0)),
        pl.BlockSpec((16, 64), lambda i: (0, 0)),
        pl.BlockSpec((1, 64), lambda i: (0, 0)),
        pl.BlockSpec((64, 128), lambda i: (0, 0)),
        pl.BlockSpec((1, 128), lambda i: (0, 0)),
    ]
    return pl.pallas_call(
        body, grid=(nb,), in_specs=in_specs,
        out_specs=pl.BlockSpec((_EB, 128), lambda i: (i, 0)),
        out_shape=jax.ShapeDtypeStruct((_E, 128), _F32))(ea, we, be, woc, bo)


def _pq_call(h2, woa, wob):
    def body(h_ref, wa_ref, wb_ref, p_ref, q_ref):
        h = h_ref[...]
        p_ref[...] = _dot(h, wa_ref[...])
        q_ref[...] = _dot(h, wb_ref[...])
    return pl.pallas_call(
        body,
        out_shape=[jax.ShapeDtypeStruct((_N, 128), _F32)] * 2)(h2, woa, wob)


def _out_call(s_arr, r_arr, wf, bf):
    nb = _E // _EB

    def body(s_ref, r_ref, wf_ref, bf_ref, out_ref):
        ef = jnp.maximum(s_ref[...] + r_ref[...], 0.0)
        out_ref[...] = _dot(ef, wf_ref[...]) + bf_ref[...]

    in_specs = [
        pl.BlockSpec((_EB, 128), lambda i: (i, 0)),
        pl.BlockSpec((_EB, 128), lambda i: (i, 0)),
        pl.BlockSpec((128, 1), lambda i: (0, 0)),
        pl.BlockSpec((1, 1), lambda i: (0, 0)),
    ]
    return pl.pallas_call(
        body, grid=(nb,), in_specs=in_specs,
        out_specs=pl.BlockSpec((_EB, 1), lambda i: (i, 0)),
        out_shape=jax.ShapeDtypeStruct((_E, 1), _F32))(s_arr, r_arr, wf, bf)


def kernel(x, edge_index, edge_attr, batch, W1, b1, W2, b2, We, be, Wo, bo,
           Wf, bf):
    x = x.astype(_F32)
    ei = edge_index.astype(jnp.int32)
    src3 = ei[0].reshape(_NW, _NCH, _CH)
    dst3 = ei[1].reshape(_NW, _NCH, _CH)
    src4 = ei[0].reshape(_NW * _NB, _BCH, _CH)
    dst4 = ei[1].reshape(_NW * _NB, _BCH, _CH)
    b2d = batch.astype(jnp.int32).reshape(_N, 1)
    zeros128 = jnp.zeros((_RT, 128), _F32)
    zeros8 = jnp.zeros((_RT, _DW), _F32)
    ones8 = jnp.ones((_CH, _DW), _F32)

    mt = _mt_call(b2d)
    degparts = _deg_sc(dst3, zeros8, ones8).reshape(2 * _N, _DW)
    deg = _degsum_call(degparts)

    g2a = _g2_call(mt, x, W1[128:])
    zlo, zhi = _z_call(x, mt, deg, W1[:128], g2a, split=True)
    agg_lo = _agg_sc(zlo, src4, dst4, zeros128).reshape(2 * _N, 128)
    agg_hi = _agg_sc(zhi, src4, dst4, zeros128).reshape(2 * _N, 128)
    h1 = _h1_call(agg_lo, agg_hi, zlo, zhi, deg, b1.reshape(1, 256))

    g2b = _g2_call(mt, h1, W2[256:])
    z2 = _z_call(h1, mt, deg, W2[:256], g2b, split=False)
    agg2 = _agg_sc(z2, src4, dst4, zeros128).reshape(2 * _N, 128)
    h2 = _h2_call(agg2, z2, deg, b2.reshape(1, 128))

    r_arr = _r_call(edge_attr.astype(_F32), We, be.reshape(1, 64),
                    Wo[256:], bo.reshape(1, 128))
    p_arr, q_arr = _pq_call(h2, Wo[:128], Wo[128:256])
    s_arr = _head_sc(p_arr, q_arr, src3, dst3).reshape(_E, 128)
    return _out_call(s_arr, r_arr, Wf, bf.reshape(1, 1))
